# baseline (speedup 1.0000x reference)
import functools

import jax
import jax.numpy as jnp
from jax import lax
from jax.experimental import pallas as pl
from jax.experimental.pallas import tpu as pltpu
from jax.experimental.pallas import tpu_sc as plsc

N = 10000
E = 320000
D = 128
H = 2 * D

NC = 2
NS = 16
NW = NC * NS
EW = E // NW
CHUNK = 80
NCHUNK = EW // CHUNK
RPT = 624
REM = N - NS * RPT


NBUF = 4
EOFF = 2
GOFF = 1
DLAG = NBUF - EOFF
BB = 16
UNROLL = 2 * BB
NBLK = (NCHUNK + BB - 1) // BB
CPAD = NBLK * BB


def _sc_segment_sum(node_feat, edge_feat, src2, dst2, zeros):
    mesh = plsc.VectorSubcoreMesh(core_axis_name="c", subcore_axis_name="s")

    @functools.partial(
        pl.kernel,
        mesh=mesh,
        out_type=jax.ShapeDtypeStruct((NC, N, D), jnp.float32),
        scratch_types=[
            [pltpu.VMEM((BB, CHUNK), jnp.int32)] * 2,
            [pltpu.VMEM((BB, CHUNK), jnp.int32)] * 2,
            [pltpu.VMEM((CHUNK, D), jnp.float32)] * NBUF,
            [pltpu.SemaphoreType.DMA] * 2,
            [pltpu.SemaphoreType.DMA] * NBUF,
            [pltpu.SemaphoreType.DMA] * NBUF,
            [pltpu.SemaphoreType.DMA] * NBUF,
            pltpu.VMEM_SHARED((N, D), jnp.float32),
        ],
    )
    def body(node_hbm, ef_hbm, src_hbm, dst_hbm, zero_hbm, out_hbm,
             sidx_v, didx_v, mrows_v, bsems, lsems, gsems, ssems, acc):
        cid = lax.axis_index("c")
        sid = lax.axis_index("s")
        wid = cid * NS + sid

        pltpu.sync_copy(zero_hbm.at[pl.ds(sid * RPT, RPT)],
                        acc.at[pl.ds(sid * RPT, RPT)])

        @pl.when(sid == NS - 1)
        def _():
            pltpu.sync_copy(zero_hbm.at[pl.ds(NS * RPT, REM)],
                            acc.at[pl.ds(NS * RPT, REM)])

        plsc.subcore_barrier()

        def issue_blk(t, p):
            row = wid * NBLK + t
            pltpu.async_copy(src_hbm.at[row], sidx_v[p], bsems[p])
            pltpu.async_copy(dst_hbm.at[row], didx_v[p], bsems[p])

        def wait_blk(p):
            pltpu.make_async_copy(src_hbm.at[0], sidx_v[p],
                                  bsems[p]).wait()
            pltpu.make_async_copy(src_hbm.at[0], didx_v[p],
                                  bsems[p]).wait()

        def issue_eload(c, b):
            base = wid * EW + c * CHUNK
            pltpu.async_copy(ef_hbm.at[pl.ds(base, CHUNK)], mrows_v[b],
                             lsems[b])

        def wait_eload(b):
            pltpu.make_async_copy(ef_hbm.at[pl.ds(0, CHUNK)],
                                  mrows_v[b], lsems[b]).wait()

        def issue_gadd(b, p, r):
            pltpu.async_copy(node_hbm.at[sidx_v[p].at[r]], mrows_v[b],
                             gsems[b], add=True)

        def wait_gadd(b):
            pltpu.make_async_copy(ef_hbm.at[pl.ds(0, CHUNK)],
                                  mrows_v[b], gsems[b]).wait()

        def issue_scatter(b, p, r):
            pltpu.async_copy(mrows_v[b], acc.at[didx_v[p].at[r]], ssems[b],
                             add=True)

        def wait_scatter(b):
            pltpu.make_async_copy(ef_hbm.at[pl.ds(0, CHUNK)],
                                  mrows_v[b], ssems[b]).wait()

        issue_blk(0, 0)
        wait_blk(0)
        for k in range(EOFF):
            issue_eload(k, k % NBUF)
        for k in range(GOFF):
            wait_eload(k % NBUF)
            issue_gadd(k % NBUF, 0, k)

        def group_body(g, carry):
            for k in range(UNROLL):
                c = g * UNROLL + k
                b = k % NBUF

                @pl.when(c < NCHUNK)
                def _():
                    be = (k + EOFF) % NBUF
                    bg = (k + GOFF) % NBUF
                    pg = ((k + GOFF) // BB) % 2
                    rg = (k + GOFF) % BB
                    pc = (k // BB) % 2
                    rc = k % BB

                    @pl.when(c + EOFF < NCHUNK)
                    def _():
                        @pl.when(c >= DLAG)
                        def _():
                            wait_scatter(be)
                        issue_eload(c + EOFF, be)

                    if k % BB == 1:
                        t = (c + BB - 1) // BB

                        @pl.when(t * BB < NCHUNK)
                        def _():
                            issue_blk(t, (pc + 1) % 2)

                    if k % BB == BB - 1:
                        t = (c + 1) // BB

                        @pl.when(t * BB < NCHUNK)
                        def _():
                            wait_blk((pc + 1) % 2)

                    @pl.when(c + GOFF < NCHUNK)
                    def _():
                        wait_eload(bg)
                        issue_gadd(bg, pg, rg)

                    wait_gadd(b)
                    issue_scatter(b, pc, rc)

            return carry

        lax.fori_loop(0, (NCHUNK + UNROLL - 1) // UNROLL, group_body, 0)

        for b in range(NBUF):
            wait_scatter(b)

        plsc.subcore_barrier()
        pltpu.sync_copy(acc.at[pl.ds(sid * RPT, RPT)],
                        out_hbm.at[cid, pl.ds(sid * RPT, RPT)])

        @pl.when(sid == NS - 1)
        def _():
            pltpu.sync_copy(acc.at[pl.ds(NS * RPT, REM)],
                            out_hbm.at[cid, pl.ds(NS * RPT, REM)])

    return body(node_feat, edge_feat, src2, dst2, zeros)


BN = 1000


def _mlp_body(agg_ref, w1_ref, b1_ref, w2_ref, b2_ref, out_ref):
    a = agg_ref[0] + agg_ref[1]
    h = jnp.maximum(
        jnp.dot(a, w1_ref[...], preferred_element_type=jnp.float32)
        + b1_ref[...], 0.0)
    out_ref[...] = (
        jnp.dot(h, w2_ref[...], preferred_element_type=jnp.float32)
        + b2_ref[...])


def _mlp(partials, W1, b1, W2, b2):
    return pl.pallas_call(
        _mlp_body,
        grid=(N // BN,),
        in_specs=[
            pl.BlockSpec((NC, BN, D), lambda i: (0, i, 0)),
            pl.BlockSpec((D, H), lambda i: (0, 0)),
            pl.BlockSpec((1, H), lambda i: (0, 0)),
            pl.BlockSpec((H, D), lambda i: (0, 0)),
            pl.BlockSpec((1, D), lambda i: (0, 0)),
        ],
        out_specs=pl.BlockSpec((BN, D), lambda i: (i, 0)),
        out_shape=jax.ShapeDtypeStruct((N, D), jnp.float32),
    )(partials, W1, b1, W2, b2)


@jax.jit
def kernel(node_feat, edge_feat, edge_index, W1, b1, W2, b2):
    pad = CPAD * CHUNK - EW
    src2 = jnp.pad(edge_index[0].reshape(NW, EW), ((0, 0), (0, pad)))
    src2 = src2.reshape(NW * NBLK, BB, CHUNK)
    dst2 = jnp.pad(edge_index[1].reshape(NW, EW), ((0, 0), (0, pad)))
    dst2 = dst2.reshape(NW * NBLK, BB, CHUNK)
    zeros = jnp.zeros((N, D), jnp.float32)
    partials = _sc_segment_sum(node_feat, edge_feat, src2, dst2, zeros)
    return _mlp(partials, W1, b1.reshape(1, H), W2, b2.reshape(1, D))
